# baseline (speedup 1.0000x reference)
import functools

import jax
import jax.numpy as jnp
from jax import lax
from jax.experimental import pallas as pl
from jax.experimental.pallas import tpu as pltpu
from jax.experimental.pallas import tpu_sc as plsc

EMB_D = 256
N_CODES = 8192
N_IMG = 16
HW = 1024
N_PIX = N_IMG * HW
P_BLK = 1024
_BOUNDS = (0, 512, 1024, 1536, 2048, 2560, 2816,
           3072, 3584, 4096, 4608, 5120, 5632,
           6144, 6656, 7168, 7680, 8192)
_ROUND_AFTER = (2816, 5632)
_INV_NUMEL = 1.0 / float(N_PIX * EMB_D)

_SC_NC = 2
_SC_NS = 16
_NW = _SC_NC * _SC_NS
_BPW = N_PIX // _NW
_GCH = 256
_NGC = _BPW // _GCH


def _vq_argmin_kernel(x_ref, e_ref, a_ref, b_ref, idx_ref, loss_ref):
    n = pl.program_id(0)
    x2 = x_ref[0] * 2.0
    a = a_ref[...]
    run_min = jnp.full((P_BLK, 1), jnp.inf, jnp.float32)
    run_idx = jnp.zeros((P_BLK, 1), jnp.float32)
    base = lax.broadcasted_iota(jnp.int32, (P_BLK, 512), 1).astype(jnp.float32)
    for lo, hi in zip(_BOUNDS[:-1], _BOUNDS[1:]):
        w = hi - lo
        ej = e_ref[lo:hi, :]
        bj = b_ref[:, lo:hi]
        m2 = lax.dot_general(x2, ej, (((0,), (1,)), ((), ())),
                             preferred_element_type=jnp.float32)
        dist = (a + bj) - m2
        bmin = jnp.min(dist, axis=1, keepdims=True)
        bidx = jnp.min(jnp.where(dist == bmin, base[:, :w], float(N_CODES)),
                       axis=1, keepdims=True) + float(lo)
        better = bmin < run_min
        run_min = jnp.where(better, bmin, run_min)
        run_idx = jnp.where(better, bidx, run_idx)
        if hi in _ROUND_AFTER:
            run_min = run_min.astype(jnp.bfloat16).astype(jnp.float32)
    idx_ref[...] = run_idx.astype(jnp.int32)

    @pl.when(n == 0)
    def _():
        loss_ref[...] = jnp.zeros_like(loss_ref)

    loss_ref[...] += jnp.sum(run_min) * _INV_NUMEL


_vq_argmin = pl.pallas_call(
    _vq_argmin_kernel,
    grid=(N_PIX // P_BLK,),
    in_specs=[
        pl.BlockSpec((1, EMB_D, P_BLK), lambda n: (n, 0, 0)),
        pl.BlockSpec((N_CODES, EMB_D), lambda n: (0, 0)),
        pl.BlockSpec((P_BLK, 1), lambda n: (n, 0)),
        pl.BlockSpec((1, N_CODES), lambda n: (0, 0)),
    ],
    out_specs=[
        pl.BlockSpec((P_BLK, 1), lambda n: (n, 0)),
        pl.BlockSpec((1, 1), lambda n: (0, 0)),
    ],
    out_shape=[
        jax.ShapeDtypeStruct((N_PIX, 1), jnp.int32),
        jax.ShapeDtypeStruct((1, 1), jnp.float32),
    ],
    compiler_params=pltpu.CompilerParams(
        dimension_semantics=("arbitrary",),
    ),
)


@functools.cache
def _make_sc_gather():
    mesh = plsc.VectorSubcoreMesh(core_axis_name="c", subcore_axis_name="s")

    @functools.partial(
        pl.kernel,
        mesh=mesh,
        out_type=jax.ShapeDtypeStruct((N_PIX, EMB_D), jnp.float32),
        scratch_types=[
            pltpu.VMEM((_NGC, _GCH), jnp.int32),
            pltpu.VMEM((_GCH, EMB_D), jnp.float32),
            pltpu.SemaphoreType.DMA,
        ],
        compiler_params=pltpu.CompilerParams(use_tc_tiling_on_sc=False),
    )
    def gather_k(table_hbm, idx_hbm, out_hbm, idx_v, rows_v, sem):
        wid = lax.axis_index("s") * _SC_NC + lax.axis_index("c")
        base = wid * _BPW
        pltpu.sync_copy(idx_hbm.at[wid], idx_v)
        for j in range(_NGC):
            pltpu.async_copy(table_hbm.at[idx_v.at[j]], rows_v, sem).wait()
            pltpu.sync_copy(rows_v, out_hbm.at[pl.ds(base + j * _GCH, _GCH)])

    return gather_k


def kernel(inputs, embedding_weight):
    flat = jnp.transpose(inputs, (0, 2, 3, 1)).reshape(-1, EMB_D)
    a = jnp.sum(flat ** 2, axis=1).reshape(N_PIX, 1)
    b = jnp.sum(embedding_weight ** 2, axis=1).reshape(1, N_CODES)
    xc = inputs.reshape(N_IMG, EMB_D, HW)
    idx2, loss2 = _vq_argmin(xc, embedding_weight, a, b)
    idx_w = idx2.reshape(_NW, _NGC, _GCH)
    q = _make_sc_gather()(embedding_weight, idx_w)
    out = jnp.transpose(q.reshape(N_IMG, 32, 32, EMB_D), (0, 3, 1, 2))
    return out, loss2[0, 0]


# speedup vs baseline: 1.0786x; 1.0786x over previous
import functools

import jax
import jax.numpy as jnp
from jax import lax
from jax.experimental import pallas as pl
from jax.experimental.pallas import tpu as pltpu
from jax.experimental.pallas import tpu_sc as plsc

EMB_D = 256
N_CODES = 8192
N_IMG = 16
HW = 1024
N_PIX = N_IMG * HW
P_BLK = 1024
_BOUNDS = (0, 512, 1024, 1536, 2048, 2560, 2816,
           3072, 3584, 4096, 4608, 5120, 5632,
           6144, 6656, 7168, 7680, 8192)
_ROUND_AFTER = (2816, 5632)
_INV_NUMEL = 1.0 / float(N_PIX * EMB_D)

_SC_NC = 2
_SC_NS = 16
_NW = _SC_NC * _SC_NS
_BPW = N_PIX // _NW
_GCH = 256
_NGC = _BPW // _GCH


def _vq_argmin_kernel(x_ref, e_ref, a_ref, b_ref, idx_ref, loss_ref):
    n = pl.program_id(0)
    x2 = x_ref[...] * 2.0
    a = a_ref[...]
    run_min = jnp.full((P_BLK, 1), jnp.inf, jnp.float32)
    run_idx = jnp.zeros((P_BLK, 1), jnp.float32)
    base = lax.broadcasted_iota(jnp.int32, (P_BLK, 512), 1).astype(jnp.float32)
    for lo, hi in zip(_BOUNDS[:-1], _BOUNDS[1:]):
        w = hi - lo
        ej = e_ref[lo:hi, :]
        bj = b_ref[:, lo:hi]
        m2 = lax.dot_general(x2, ej, (((1,), (1,)), ((), ())),
                             preferred_element_type=jnp.float32)
        dist = (a + bj) - m2
        bmin = jnp.min(dist, axis=1, keepdims=True)
        bidx = jnp.min(jnp.where(dist == bmin, base[:, :w], float(N_CODES)),
                       axis=1, keepdims=True) + float(lo)
        better = bmin < run_min
        run_min = jnp.where(better, bmin, run_min)
        run_idx = jnp.where(better, bidx, run_idx)
        if hi in _ROUND_AFTER:
            run_min = run_min.astype(jnp.bfloat16).astype(jnp.float32)
    idx_ref[...] = run_idx.astype(jnp.int32)

    @pl.when(n == 0)
    def _():
        loss_ref[...] = jnp.zeros_like(loss_ref)

    loss_ref[...] += jnp.sum(run_min) * _INV_NUMEL


_vq_argmin = pl.pallas_call(
    _vq_argmin_kernel,
    grid=(N_PIX // P_BLK,),
    in_specs=[
        pl.BlockSpec((P_BLK, EMB_D), lambda n: (n, 0)),
        pl.BlockSpec((N_CODES, EMB_D), lambda n: (0, 0)),
        pl.BlockSpec((P_BLK, 1), lambda n: (n, 0)),
        pl.BlockSpec((1, N_CODES), lambda n: (0, 0)),
    ],
    out_specs=[
        pl.BlockSpec((P_BLK, 1), lambda n: (n, 0)),
        pl.BlockSpec((1, 1), lambda n: (0, 0)),
    ],
    out_shape=[
        jax.ShapeDtypeStruct((N_PIX, 1), jnp.int32),
        jax.ShapeDtypeStruct((1, 1), jnp.float32),
    ],
    compiler_params=pltpu.CompilerParams(
        dimension_semantics=("arbitrary",),
    ),
)


@functools.cache
def _make_sc_gather():
    mesh = plsc.VectorSubcoreMesh(core_axis_name="c", subcore_axis_name="s")

    @functools.partial(
        pl.kernel,
        mesh=mesh,
        out_type=jax.ShapeDtypeStruct((N_PIX, EMB_D), jnp.float32),
        scratch_types=[
            pltpu.VMEM((_NGC, _GCH), jnp.int32),
            pltpu.VMEM((_GCH, EMB_D), jnp.float32),
            pltpu.SemaphoreType.DMA,
        ],
        compiler_params=pltpu.CompilerParams(use_tc_tiling_on_sc=False),
    )
    def gather_k(table_hbm, idx_hbm, out_hbm, idx_v, rows_v, sem):
        wid = lax.axis_index("s") * _SC_NC + lax.axis_index("c")
        base = wid * _BPW
        pltpu.sync_copy(idx_hbm.at[wid], idx_v)
        for j in range(_NGC):
            pltpu.async_copy(table_hbm.at[idx_v.at[j]], rows_v, sem).wait()
            pltpu.sync_copy(rows_v, out_hbm.at[pl.ds(base + j * _GCH, _GCH)])

    return gather_k


def kernel(inputs, embedding_weight):
    flat = jnp.transpose(inputs, (0, 2, 3, 1)).reshape(-1, EMB_D)
    a = jnp.sum(flat ** 2, axis=1).reshape(N_PIX, 1)
    b = jnp.sum(embedding_weight ** 2, axis=1).reshape(1, N_CODES)
    idx2, loss2 = _vq_argmin(flat, embedding_weight, a, b)
    idx_w = idx2.reshape(_NW, _NGC, _GCH)
    q = _make_sc_gather()(embedding_weight, idx_w)
    out = jnp.transpose(q.reshape(N_IMG, 32, 32, EMB_D), (0, 3, 1, 2))
    return out, loss2[0, 0]
